# baseline (speedup 1.0000x reference)
import functools

import jax
import jax.numpy as jnp
from jax import lax
from jax.experimental import pallas as pl
from jax.experimental.pallas import tpu as pltpu
from jax.experimental.pallas import tpu_sc as plsc

NC, NS, L = 2, 16, 16
NW = NC * NS
B = 2
C = 128
CG = C // L
NCLS = 2
N = 32 * 32 * 32
SC_N = 8192
TC_N = N - SC_N
PPW = SC_N // NW
PBLK = 128
NBLK = PPW // PBLK
NSTEP = B * NBLK
GPB = PBLK // L
TP = 2048
NTBLK = TC_N // TP


def _masked_sqrt(x):
    xc = jnp.maximum(x, jnp.float32(1e-12))
    i = lax.bitcast_convert_type(xc, jnp.int32)
    seed = jnp.full((L,), 0x5F3759DF, jnp.int32) - (i >> 1)
    y = lax.bitcast_convert_type(seed, jnp.float32)
    for _ in range(3):
        y = y * (jnp.float32(1.5) - jnp.float32(0.5) * xc * y * y)
    return x * y


def _sc_body(feat, tgt, protos, out, fb0, fb1, tgtv, pv, stg, outv, sem0, sem1):
    wid = lax.axis_index("s") * NC + lax.axis_index("c")
    base = wid * PPW
    pltpu.sync_copy(protos, pv)
    for b in range(B):
        pltpu.sync_copy(tgt.at[b, pl.ds(base, PPW)], tgtv.at[pl.ds(b * PPW, PPW)])
    zero = jnp.zeros((L,), jnp.float32)
    one = jnp.ones((L,), jnp.float32)
    for j in range(2 * NCLS * B):
        outv[j, :] = zero

    p0 = [pv[0, pl.ds(j * L, L)] for j in range(CG)]
    p1 = [pv[1, pl.ds(j * L, L)] for j in range(CG)]

    def start(step, buf, sem):
        b = step // NBLK
        blk = lax.rem(step, NBLK)
        pltpu.async_copy(
            feat.at[b, pl.ds(base + blk * PBLK, PBLK), :], buf, sem)

    start(0, fb0, sem0)
    start(1, fb1, sem1)

    def process(step, buf, sem):
        b = step // NBLK
        blk = lax.rem(step, NBLK)
        pltpu.make_async_copy(feat.at[0, pl.ds(0, PBLK), :], buf, sem).wait()

        def gbody(g, carry):
            s0, c0, s1, c1 = carry
            tvec = tgtv[pl.ds(b * PPW + blk * PBLK + g * L, L)]
            for p in range(L):
                pix = g * L + p
                tsp = jnp.take_along_axis(
                    tvec, jnp.full((L,), p, jnp.int32), axis=0)
                m = tsp == 0
                acc_a = zero
                acc_b = zero
                for j in range(CG):
                    v = buf[pix, pl.ds(j * L, L)]
                    psel = jnp.where(m, p0[j], p1[j])
                    d = v - psel
                    if j % 2 == 0:
                        acc_a = acc_a + d * d
                    else:
                        acc_b = acc_b + d * d
                stg[pl.ds(p * L, L)] = acc_a + acc_b
            rowbase = lax.iota(jnp.int32, L) * L
            d2 = zero
            for j in range(L):
                d2 = d2 + plsc.load_gather(stg, [rowbase + j])
            m0 = tvec == 0
            m1 = tvec == 1
            dist = _masked_sqrt(d2)
            s0 = s0 + jnp.where(m0, dist, zero)
            c0 = c0 + jnp.where(m0, one, zero)
            s1 = s1 + jnp.where(m1, dist, zero)
            c1 = c1 + jnp.where(m1, one, zero)
            return (s0, c0, s1, c1)

        s0, c0, s1, c1 = lax.fori_loop(
            0, GPB, gbody, (zero, zero, zero, zero), unroll=1)

        @pl.when(step + 2 < NSTEP)
        def _():
            start(step + 2, buf, sem)

        row = b * 4
        outv[row, :] = outv[row, :] + s0
        outv[row + 1, :] = outv[row + 1, :] + c0
        outv[row + 2, :] = outv[row + 2, :] + s1
        outv[row + 3, :] = outv[row + 3, :] + c1

    def loop_body(j, carry):
        process(2 * j, fb0, sem0)
        process(2 * j + 1, fb1, sem1)
        return carry

    lax.fori_loop(0, NSTEP // 2, loop_body, jnp.int32(0), unroll=1)
    pltpu.sync_copy(outv, out.at[wid])


@functools.partial(jax.jit)
def _sc_call(feat, tgt, protos):
    mesh = plsc.VectorSubcoreMesh(core_axis_name="c", subcore_axis_name="s")
    return pl.kernel(
        _sc_body,
        out_type=jax.ShapeDtypeStruct((NW, 2 * NCLS * B, L), jnp.float32),
        mesh=mesh,
        compiler_params=pltpu.CompilerParams(needs_layout_passes=False),
        scratch_types=[
            pltpu.VMEM((PBLK, C), jnp.float32),
            pltpu.VMEM((PBLK, C), jnp.float32),
            pltpu.VMEM((B * PPW,), jnp.int32),
            pltpu.VMEM((NCLS, C), jnp.float32),
            pltpu.VMEM((L * L,), jnp.float32),
            pltpu.VMEM((2 * NCLS * B, L), jnp.float32),
            pltpu.SemaphoreType.DMA,
            pltpu.SemaphoreType.DMA,
        ],
    )(feat, tgt, protos)


def _tc_body(f_ref, t_ref, p_ref, o_ref):
    f = f_ref[0]
    t = t_ref[0]
    p0 = p_ref[0, :][None, :]
    p1 = p_ref[1, :][None, :]
    m0 = t == 0
    m1 = t == 1
    psel = jnp.where(m0, p0, p1)
    d = f - psel
    d2 = jnp.sum(d * d, axis=1, keepdims=True)
    dist = jnp.sqrt(d2)
    zero = jnp.zeros_like(dist)
    s0 = jnp.sum(jnp.where(m0, dist, zero))
    c0 = jnp.sum(m0.astype(jnp.float32))
    s1 = jnp.sum(jnp.where(m1, dist, zero))
    c1 = jnp.sum(m1.astype(jnp.float32))

    def row(x):
        return jnp.full((1, C), x, jnp.float32)

    o_ref[0] = jnp.concatenate([row(s0), row(c0), row(s1), row(c1)], axis=0)


@functools.partial(jax.jit)
def _tc_call(feat, tgt3, protos):
    return pl.pallas_call(
        _tc_body,
        grid=(B, NTBLK),
        in_specs=[
            pl.BlockSpec((1, TP, C), lambda b, i: (b, SC_N // TP + i, 0)),
            pl.BlockSpec((1, TP, 1),
                         lambda b, i: (b * (N // TP) + SC_N // TP + i, 0, 0)),
            pl.BlockSpec((NCLS, C), lambda b, i: (0, 0)),
        ],
        out_specs=pl.BlockSpec((1, 4, C), lambda b, i: (b * NTBLK + i, 0, 0)),
        out_shape=jax.ShapeDtypeStruct((B * NTBLK, 4, C), jnp.float32),
    )(feat, tgt3, protos)


def kernel(features, predictions, targets, prototypes):
    del predictions
    feat = jnp.transpose(features, (0, 2, 3, 4, 1)).reshape(B, N, C)
    tgt = targets.reshape(B, N)
    part_sc = _sc_call(feat, tgt, prototypes)
    tgt3 = tgt.reshape(B * (N // TP), TP, 1)
    part_tc = _tc_call(feat, tgt3, prototypes)
    sums_sc = part_sc.reshape(NW, B, NCLS, 2, L).sum(axis=(0, 4))
    sums_tc = part_tc[:, :, 0].reshape(B, NTBLK, NCLS, 2).sum(axis=1)
    sums = sums_sc + sums_tc
    s = sums[..., 0]
    n = sums[..., 1]
    mean = jnp.where(n > 0, s / jnp.maximum(n, 1.0), 0.0)
    total = mean.sum()
    valid = (n > 0).astype(jnp.float32).sum()
    return jnp.where(valid > 0, total / valid, jnp.float32(0.0))


# speedup vs baseline: 1.6713x; 1.6713x over previous
import functools

import jax
import jax.numpy as jnp
from jax import lax
from jax.experimental import pallas as pl
from jax.experimental.pallas import tpu as pltpu
from jax.experimental.pallas import tpu_sc as plsc

NC, NS, L = 2, 16, 16
NW = NC * NS
B = 2
C = 128
CG = C // L
NCLS = 2
N = 32 * 32 * 32
SC_N = 8192
TC_N = N - SC_N
PPW = SC_N // NW
PBLK = 128
NBLK = PPW // PBLK
NSTEP = B * NBLK
GPB = PBLK // L
TP = 2048
NTBLK = TC_N // TP


def _masked_sqrt(x):
    xc = jnp.maximum(x, jnp.float32(1e-12))
    i = lax.bitcast_convert_type(xc, jnp.int32)
    seed = jnp.full((L,), 0x5F3759DF, jnp.int32) - (i >> 1)
    y = lax.bitcast_convert_type(seed, jnp.float32)
    for _ in range(3):
        y = y * (jnp.float32(1.5) - jnp.float32(0.5) * xc * y * y)
    return x * y


def _sc_body(feat, tgt, protos, out, fb0, fb1, tgtv, pv, stg, outv, sem0, sem1):
    wid = lax.axis_index("s") * NC + lax.axis_index("c")
    base = wid * PPW
    pltpu.sync_copy(protos, pv)
    for b in range(B):
        pltpu.sync_copy(tgt.at[b, pl.ds(base, PPW)], tgtv.at[pl.ds(b * PPW, PPW)])
    zero = jnp.zeros((L,), jnp.float32)
    one = jnp.ones((L,), jnp.float32)
    for j in range(2 * NCLS * B):
        outv[j, :] = zero

    p0 = [pv[0, pl.ds(j * L, L)] for j in range(CG)]
    p1 = [pv[1, pl.ds(j * L, L)] for j in range(CG)]

    def start(step, buf, sem):
        b = step // NBLK
        blk = lax.rem(step, NBLK)
        pltpu.async_copy(
            feat.at[b, pl.ds(base + blk * PBLK, PBLK), :], buf, sem)

    start(0, fb0, sem0)
    start(1, fb1, sem1)

    def process(step, buf, sem):
        b = step // NBLK
        blk = lax.rem(step, NBLK)
        pltpu.make_async_copy(feat.at[0, pl.ds(0, PBLK), :], buf, sem).wait()

        def gbody(g, carry):
            s0, c0, s1, c1 = carry
            tvec = tgtv[pl.ds(b * PPW + blk * PBLK + g * L, L)]
            for p in range(L):
                pix = g * L + p
                tsp = jnp.take_along_axis(
                    tvec, jnp.full((L,), p, jnp.int32), axis=0)
                m = tsp == 0
                acc_a = zero
                acc_b = zero
                for j in range(CG):
                    v = buf[pix, pl.ds(j * L, L)]
                    psel = jnp.where(m, p0[j], p1[j])
                    d = v - psel
                    if j % 2 == 0:
                        acc_a = acc_a + d * d
                    else:
                        acc_b = acc_b + d * d
                stg[pl.ds(p * L, L)] = acc_a + acc_b
            rowbase = lax.iota(jnp.int32, L) * L
            d2 = zero
            for j in range(L):
                d2 = d2 + plsc.load_gather(stg, [rowbase + j])
            m0 = tvec == 0
            m1 = tvec == 1
            dist = _masked_sqrt(d2)
            s0 = s0 + jnp.where(m0, dist, zero)
            c0 = c0 + jnp.where(m0, one, zero)
            s1 = s1 + jnp.where(m1, dist, zero)
            c1 = c1 + jnp.where(m1, one, zero)
            return (s0, c0, s1, c1)

        s0, c0, s1, c1 = lax.fori_loop(
            0, GPB, gbody, (zero, zero, zero, zero), unroll=1)

        @pl.when(step + 2 < NSTEP)
        def _():
            start(step + 2, buf, sem)

        row = b * 4
        outv[row, :] = outv[row, :] + s0
        outv[row + 1, :] = outv[row + 1, :] + c0
        outv[row + 2, :] = outv[row + 2, :] + s1
        outv[row + 3, :] = outv[row + 3, :] + c1

    def loop_body(j, carry):
        process(2 * j, fb0, sem0)
        process(2 * j + 1, fb1, sem1)
        return carry

    lax.fori_loop(0, NSTEP // 2, loop_body, jnp.int32(0), unroll=1)
    pltpu.sync_copy(outv, out.at[wid])


@functools.partial(jax.jit)
def _sc_call(feat, tgt, protos):
    mesh = plsc.VectorSubcoreMesh(core_axis_name="c", subcore_axis_name="s")
    return pl.kernel(
        _sc_body,
        out_type=jax.ShapeDtypeStruct((NW, 2 * NCLS * B, L), jnp.float32),
        mesh=mesh,
        compiler_params=pltpu.CompilerParams(needs_layout_passes=False),
        scratch_types=[
            pltpu.VMEM((PBLK, C), jnp.float32),
            pltpu.VMEM((PBLK, C), jnp.float32),
            pltpu.VMEM((B * PPW,), jnp.int32),
            pltpu.VMEM((NCLS, C), jnp.float32),
            pltpu.VMEM((L * L,), jnp.float32),
            pltpu.VMEM((2 * NCLS * B, L), jnp.float32),
            pltpu.SemaphoreType.DMA,
            pltpu.SemaphoreType.DMA,
        ],
    )(feat, tgt, protos)


def _tc_body(f_ref, t_ref, pt_ref, pp_ref, o_ref):
    f = f_ref[0]
    t = t_ref[0]
    pt = pt_ref[...]
    pp = pp_ref[...]
    dot = jnp.dot(f, pt, preferred_element_type=jnp.float32)
    s = jnp.dot(f * f, jnp.ones((C, 1), jnp.float32),
                preferred_element_type=jnp.float32)
    d2 = jnp.maximum(s - 2.0 * dot + pp, 0.0)
    dist = jnp.sqrt(d2)
    mf0 = (t == 0).astype(jnp.float32)
    mf1 = (t == 1).astype(jnp.float32)
    onecol = jnp.ones((TP, 1), jnp.float32)
    s0 = jnp.sum(jnp.dot(mf0, dist[:, 0:1], preferred_element_type=jnp.float32))
    s1 = jnp.sum(jnp.dot(mf1, dist[:, 1:2], preferred_element_type=jnp.float32))
    c0 = jnp.sum(jnp.dot(mf0, onecol, preferred_element_type=jnp.float32))
    c1 = jnp.sum(jnp.dot(mf1, onecol, preferred_element_type=jnp.float32))

    def row(x):
        return jnp.full((1, C), x, jnp.float32)

    o_ref[0] = jnp.concatenate([row(s0), row(c0), row(s1), row(c1)], axis=0)


@functools.partial(jax.jit)
def _tc_call(feat, tgtr, protosT, pp):
    return pl.pallas_call(
        _tc_body,
        grid=(B, NTBLK),
        in_specs=[
            pl.BlockSpec((1, TP, C), lambda b, i: (b, SC_N // TP + i, 0)),
            pl.BlockSpec((1, 1, TP),
                         lambda b, i: (b * (N // TP) + SC_N // TP + i, 0, 0)),
            pl.BlockSpec((C, NCLS), lambda b, i: (0, 0)),
            pl.BlockSpec((1, NCLS), lambda b, i: (0, 0)),
        ],
        out_specs=pl.BlockSpec((1, 4, C), lambda b, i: (b * NTBLK + i, 0, 0)),
        out_shape=jax.ShapeDtypeStruct((B * NTBLK, 4, C), jnp.float32),
    )(feat, tgtr, protosT, pp)


def kernel(features, predictions, targets, prototypes):
    del predictions
    feat = jnp.transpose(features, (0, 2, 3, 4, 1)).reshape(B, N, C)
    tgt = targets.reshape(B, N)
    part_sc = _sc_call(feat, tgt, prototypes)
    tgtr = tgt.reshape(B * (N // TP), 1, TP)
    protosT = prototypes.T
    pp = jnp.sum(prototypes * prototypes, axis=1)[None, :]
    part_tc = _tc_call(feat, tgtr, protosT, pp)
    sums_sc = part_sc.reshape(NW, B, NCLS, 2, L).sum(axis=(0, 4))
    sums_tc = part_tc[:, :, 0].reshape(B, NTBLK, NCLS, 2).sum(axis=1)
    sums = sums_sc + sums_tc
    s = sums[..., 0]
    n = sums[..., 1]
    mean = jnp.where(n > 0, s / jnp.maximum(n, 1.0), 0.0)
    total = mean.sum()
    valid = (n > 0).astype(jnp.float32).sum()
    return jnp.where(valid > 0, total / valid, jnp.float32(0.0))


# speedup vs baseline: 1.9597x; 1.1725x over previous
import functools

import jax
import jax.numpy as jnp
from jax import lax
from jax.experimental import pallas as pl
from jax.experimental.pallas import tpu as pltpu
from jax.experimental.pallas import tpu_sc as plsc

NC, NS, L = 2, 16, 16
NW = NC * NS
B = 2
C = 128
CG = C // L
NCLS = 2
N = 32 * 32 * 32
SC_N = 8192
TC_N = N - SC_N
PPW = SC_N // NW
PBLK = 128
NBLK = PPW // PBLK
NSTEP = B * NBLK
GPB = PBLK // L
TP = 4096
NTBLK = TC_N // TP


def _masked_sqrt(x):
    xc = jnp.maximum(x, jnp.float32(1e-12))
    i = lax.bitcast_convert_type(xc, jnp.int32)
    seed = jnp.full((L,), 0x5F3759DF, jnp.int32) - (i >> 1)
    y = lax.bitcast_convert_type(seed, jnp.float32)
    for _ in range(3):
        y = y * (jnp.float32(1.5) - jnp.float32(0.5) * xc * y * y)
    return x * y


def _sc_body(feat, tgt, protos, out, fb0, fb1, tgtv, pv, stg, outv, sem0, sem1):
    wid = lax.axis_index("s") * NC + lax.axis_index("c")
    base = wid * PPW
    pltpu.sync_copy(protos, pv)
    for b in range(B):
        pltpu.sync_copy(tgt.at[b, pl.ds(base, PPW)], tgtv.at[pl.ds(b * PPW, PPW)])
    zero = jnp.zeros((L,), jnp.float32)
    one = jnp.ones((L,), jnp.float32)
    for j in range(2 * NCLS * B):
        outv[j, :] = zero

    p0 = [pv[0, pl.ds(j * L, L)] for j in range(CG)]
    p1 = [pv[1, pl.ds(j * L, L)] for j in range(CG)]

    def start(step, buf, sem):
        b = step // NBLK
        blk = lax.rem(step, NBLK)
        pltpu.async_copy(
            feat.at[b, pl.ds(base + blk * PBLK, PBLK), :], buf, sem)

    start(0, fb0, sem0)
    start(1, fb1, sem1)

    def process(step, buf, sem):
        b = step // NBLK
        blk = lax.rem(step, NBLK)
        pltpu.make_async_copy(feat.at[0, pl.ds(0, PBLK), :], buf, sem).wait()

        def gbody(g, carry):
            s0, c0, s1, c1 = carry
            tvec = tgtv[pl.ds(b * PPW + blk * PBLK + g * L, L)]
            for p in range(L):
                pix = g * L + p
                tsp = jnp.take_along_axis(
                    tvec, jnp.full((L,), p, jnp.int32), axis=0)
                m = tsp == 0
                acc_a = zero
                acc_b = zero
                for j in range(CG):
                    v = buf[pix, pl.ds(j * L, L)]
                    psel = jnp.where(m, p0[j], p1[j])
                    d = v - psel
                    if j % 2 == 0:
                        acc_a = acc_a + d * d
                    else:
                        acc_b = acc_b + d * d
                stg[pl.ds(p * L, L)] = acc_a + acc_b
            rowbase = lax.iota(jnp.int32, L) * L
            d2 = zero
            for j in range(L):
                d2 = d2 + plsc.load_gather(stg, [rowbase + j])
            m0 = tvec == 0
            m1 = tvec == 1
            dist = _masked_sqrt(d2)
            s0 = s0 + jnp.where(m0, dist, zero)
            c0 = c0 + jnp.where(m0, one, zero)
            s1 = s1 + jnp.where(m1, dist, zero)
            c1 = c1 + jnp.where(m1, one, zero)
            return (s0, c0, s1, c1)

        s0, c0, s1, c1 = lax.fori_loop(
            0, GPB, gbody, (zero, zero, zero, zero), unroll=1)

        @pl.when(step + 2 < NSTEP)
        def _():
            start(step + 2, buf, sem)

        row = b * 4
        outv[row, :] = outv[row, :] + s0
        outv[row + 1, :] = outv[row + 1, :] + c0
        outv[row + 2, :] = outv[row + 2, :] + s1
        outv[row + 3, :] = outv[row + 3, :] + c1

    def loop_body(j, carry):
        process(2 * j, fb0, sem0)
        process(2 * j + 1, fb1, sem1)
        return carry

    lax.fori_loop(0, NSTEP // 2, loop_body, jnp.int32(0), unroll=1)
    pltpu.sync_copy(outv, out.at[wid])


@functools.partial(jax.jit)
def _sc_call(feat, tgt, protos):
    mesh = plsc.VectorSubcoreMesh(core_axis_name="c", subcore_axis_name="s")
    return pl.kernel(
        _sc_body,
        out_type=jax.ShapeDtypeStruct((NW, 2 * NCLS * B, L), jnp.float32),
        mesh=mesh,
        compiler_params=pltpu.CompilerParams(needs_layout_passes=False),
        scratch_types=[
            pltpu.VMEM((PBLK, C), jnp.float32),
            pltpu.VMEM((PBLK, C), jnp.float32),
            pltpu.VMEM((B * PPW,), jnp.int32),
            pltpu.VMEM((NCLS, C), jnp.float32),
            pltpu.VMEM((L * L,), jnp.float32),
            pltpu.VMEM((2 * NCLS * B, L), jnp.float32),
            pltpu.SemaphoreType.DMA,
            pltpu.SemaphoreType.DMA,
        ],
    )(feat, tgt, protos)


def _tc_body(f_ref, t_ref, pt_ref, pp_ref, o_ref):
    f = f_ref[0]
    t = t_ref[0]
    pt = pt_ref[...]
    pp = pp_ref[...]
    dot = jnp.dot(f, pt, preferred_element_type=jnp.float32)
    sq = jnp.dot(f * f, jnp.ones((C, 1), jnp.float32),
                 preferred_element_type=jnp.float32)
    dot_t = jnp.transpose(dot)
    sq_t = jnp.transpose(sq)
    pp0 = pp[0, 0]
    pp1 = pp[0, 1]
    d20 = jnp.maximum(sq_t - 2.0 * dot_t[0:1, :] + pp0, 0.0)
    d21 = jnp.maximum(sq_t - 2.0 * dot_t[1:2, :] + pp1, 0.0)
    dist0 = jnp.sqrt(d20)
    dist1 = jnp.sqrt(d21)
    zero = jnp.zeros_like(dist0)
    m0 = t == 0
    m1 = t == 1
    s0 = jnp.sum(jnp.where(m0, dist0, zero))
    s1 = jnp.sum(jnp.where(m1, dist1, zero))
    c0 = jnp.sum(m0.astype(jnp.float32))
    c1 = jnp.sum(m1.astype(jnp.float32))

    def row(x):
        return jnp.full((1, C), x, jnp.float32)

    o_ref[0] = jnp.concatenate([row(s0), row(c0), row(s1), row(c1)], axis=0)


@functools.partial(jax.jit)
def _tc_call(feat, tgtr, protosT, pp):
    return pl.pallas_call(
        _tc_body,
        grid=(B, NTBLK),
        in_specs=[
            pl.BlockSpec((1, TP, C), lambda b, i: (b, SC_N // TP + i, 0)),
            pl.BlockSpec((1, 1, TP),
                         lambda b, i: (b * (N // TP) + SC_N // TP + i, 0, 0)),
            pl.BlockSpec((C, NCLS), lambda b, i: (0, 0)),
            pl.BlockSpec((1, NCLS), lambda b, i: (0, 0)),
        ],
        out_specs=pl.BlockSpec((1, 4, C), lambda b, i: (b * NTBLK + i, 0, 0)),
        out_shape=jax.ShapeDtypeStruct((B * NTBLK, 4, C), jnp.float32),
    )(feat, tgtr, protosT, pp)


def kernel(features, predictions, targets, prototypes):
    del predictions
    feat = jnp.transpose(features, (0, 2, 3, 4, 1)).reshape(B, N, C)
    tgt = targets.reshape(B, N)
    part_sc = _sc_call(feat, tgt, prototypes)
    tgtr = tgt.reshape(B * (N // TP), 1, TP)
    protosT = prototypes.T
    pp = jnp.sum(prototypes * prototypes, axis=1)[None, :]
    part_tc = _tc_call(feat, tgtr, protosT, pp)
    sums_sc = part_sc.reshape(NW, B, NCLS, 2, L).sum(axis=(0, 4))
    sums_tc = part_tc[:, :, 0].reshape(B, NTBLK, NCLS, 2).sum(axis=1)
    sums = sums_sc + sums_tc
    s = sums[..., 0]
    n = sums[..., 1]
    mean = jnp.where(n > 0, s / jnp.maximum(n, 1.0), 0.0)
    total = mean.sum()
    valid = (n > 0).astype(jnp.float32).sum()
    return jnp.where(valid > 0, total / valid, jnp.float32(0.0))
